# baseline (speedup 1.0000x reference)
import functools

import jax
import jax.numpy as jnp
from jax import lax
from jax.experimental import pallas as pl
from jax.experimental.pallas import tpu as pltpu
from jax.experimental.pallas import tpu_sc as plsc

SEQ = 8192
DIM = 1024
BATCH = 4
NC = 2
NS = 16
NW = NC * NS
ROWS_PER_W = SEQ // NW
R_BLK = 16
N_BLKS = ROWS_PER_W // R_BLK
N_TASKS = N_BLKS * BATCH
NBUF = 4

_mesh = plsc.VectorSubcoreMesh(core_axis_name="c", subcore_axis_name="s")


@functools.partial(
    pl.kernel,
    mesh=_mesh,
    out_type=jax.ShapeDtypeStruct((BATCH, SEQ, DIM), jnp.float32),
    scratch_types=(
        [pltpu.VMEM((R_BLK, DIM), jnp.float32) for _ in range(2)]
        + [pltpu.VMEM((R_BLK, DIM), jnp.float32) for _ in range(NBUF)]
        + [pltpu.SemaphoreType.DMA for _ in range(2 + 2 * NBUF)]
    ),
)
def _sc_add(x_hbm, t_hbm, o_hbm, tb0, tb1, xb0, xb1, xb2, xb3, st0, st1,
            sl0, sl1, sl2, sl3, ss0, ss1, ss2, ss3):
    wid = lax.axis_index("s") * NC + lax.axis_index("c")
    row0 = wid * ROWS_PER_W
    tbuf = (tb0, tb1)
    xbuf = (xb0, xb1, xb2, xb3)
    sem_t = (st0, st1)
    sem_l = (sl0, sl1, sl2, sl3)
    sem_s = (ss0, ss1, ss2, ss3)

    def t_rows(blk):
        return pl.ds(row0 + blk * R_BLK, R_BLK)

    tload_h = [None] * N_BLKS
    load_h = [None] * N_TASKS
    store_h = [None] * N_TASKS
    tload_h[0] = pltpu.async_copy(t_hbm.at[t_rows(0)], tbuf[0], sem_t[0])
    load_h[0] = pltpu.async_copy(x_hbm.at[0, t_rows(0), :], xbuf[0], sem_l[0])
    for t in range(N_TASKS):
        blk, b = divmod(t, BATCH)
        slot = t % NBUF
        if b == 0:
            tload_h[blk].wait()
            if blk + 1 < N_BLKS:
                ts = (blk + 1) % 2
                tload_h[blk + 1] = pltpu.async_copy(
                    t_hbm.at[t_rows(blk + 1)], tbuf[ts], sem_t[ts])
        if t + 1 < N_TASKS:
            nslot = (t + 1) % NBUF
            if t + 1 >= NBUF:
                store_h[t + 1 - NBUF].wait()
            nblk, nb = divmod(t + 1, BATCH)
            load_h[t + 1] = pltpu.async_copy(
                x_hbm.at[nb, t_rows(nblk), :], xbuf[nslot], sem_l[nslot])
        load_h[t].wait()
        store_h[t] = pltpu.async_copy(
            xbuf[slot], o_hbm.at[b, t_rows(blk), :], sem_s[slot])
    for t in range(N_TASKS - NBUF, N_TASKS):
        store_h[t].wait()


def kernel(inputs, pos_table):
    return _sc_add(inputs, pos_table)


# speedup vs baseline: 1.3595x; 1.3595x over previous
import jax
import jax.numpy as jnp
from jax.experimental import pallas as pl
from jax.experimental.pallas import tpu as pltpu

S_BLK = 2048


def _add_kernel(x_ref, t_ref, o_ref):
    o_ref[0] = x_ref[0] + t_ref[...]


def kernel(inputs, pos_table):
    batch, seq, dim = inputs.shape
    grid = (seq // S_BLK, batch)
    return pl.pallas_call(
        _add_kernel,
        grid=grid,
        in_specs=[
            pl.BlockSpec((1, S_BLK, dim), lambda i, b: (b, i, 0)),
            pl.BlockSpec((S_BLK, dim), lambda i, b: (i, 0)),
        ],
        out_specs=pl.BlockSpec((1, S_BLK, dim), lambda i, b: (b, i, 0)),
        out_shape=jax.ShapeDtypeStruct(inputs.shape, inputs.dtype),
        compiler_params=pltpu.CompilerParams(
            dimension_semantics=("parallel", "parallel"),
        ),
    )(inputs, pos_table)
